# baseline (speedup 1.0000x reference)
import functools

import jax
import jax.numpy as jnp
from jax import lax
from jax.experimental import pallas as pl
from jax.experimental.pallas import tpu as pltpu
from jax.experimental.pallas import tpu_sc as plsc

N = 10000
E = 160000
D = 128
NCLASS = 16
NLAYERS = 4
STEP = 1.0
DT = 1.0
ALPHA = 1.0
GAMMA = 1.0

NC = 2
NS = 16
NW = NC * NS
CH = 128
NCH = 40
EP = NW * NCH * CH
EPCH = EP // CH
K0 = 80
KH = 40
NPA = 10112
RPTA = NPA // NS
DUMMY = 10000
NP = 10240
BN = 1024
RPT = NP // NS

_MESH = plsc.VectorSubcoreMesh(
    core_axis_name="c", subcore_axis_name="s", num_cores=NC, num_subcores=NS)


def _deg_body(dst_hbm, zeros_hbm, ones_hbm, out_hbm, acc, idx_v, ones_v, sem):
    c = lax.axis_index("c")
    s = lax.axis_index("s")
    wid = s * NC + c
    pltpu.sync_copy(zeros_hbm, acc.at[pl.ds(s * RPT, RPT)])
    pltpu.sync_copy(ones_hbm, ones_v)
    pltpu.sync_copy(dst_hbm.at[pl.ds(wid * NCH, NCH)], idx_v)
    plsc.subcore_barrier()

    def fire(j, carry):
        pltpu.async_copy(ones_v, acc.at[idx_v.at[j]], sem, add=True)
        return carry

    lax.fori_loop(0, NCH, fire, 0)

    def drain(j, carry):
        pltpu.make_async_copy(ones_v, acc.at[idx_v.at[j]], sem).wait()
        return carry

    lax.fori_loop(0, NCH, drain, 0)
    plsc.subcore_barrier()
    pltpu.sync_copy(acc.at[pl.ds(s * RPT, RPT)],
                    out_hbm.at[c].at[pl.ds(s * RPT, RPT)])


def _deg_call(dst_idx, zeros128, ones128):
    return pl.kernel(
        _deg_body,
        out_type=jax.ShapeDtypeStruct((NC, NP, D), jnp.float32),
        mesh=_MESH,
        scratch_types=[
            pltpu.VMEM_SHARED((NP, D), jnp.float32),
            pltpu.VMEM((NCH, CH), jnp.int32),
            pltpu.VMEM((CH, D), jnp.float32),
            pltpu.SemaphoreType.DMA,
        ],
    )(dst_idx, zeros128, ones128)


NB = 2


def _conv_pipeline(xs_hbm, acc, gidx, sidx, rows, gsem, ssem, K):
    ng = K // NB
    for b in range(NB):
        pltpu.async_copy(xs_hbm.at[gidx.at[b]], rows.at[b], gsem[b])

    def group(g, carry):
        for b in range(NB):
            j = g * NB + b
            pltpu.make_async_copy(
                xs_hbm.at[gidx.at[j]], rows.at[b], gsem[b]).wait()
            pltpu.async_copy(rows.at[b], acc.at[sidx.at[j]], ssem[b], add=True)
        jn = (g + 1) * NB
        for b in range(NB):
            j = g * NB + b
            pltpu.make_async_copy(
                rows.at[b], acc.at[sidx.at[j]], ssem[b]).wait()
            pltpu.async_copy(xs_hbm.at[gidx.at[jn + b]], rows.at[b], gsem[b])
        return carry

    lax.fori_loop(0, ng - 1, group, 0)

    for b in range(NB):
        j = (ng - 1) * NB + b
        pltpu.make_async_copy(xs_hbm.at[gidx.at[j]], rows.at[b], gsem[b]).wait()
        pltpu.async_copy(rows.at[b], acc.at[sidx.at[j]], ssem[b], add=True)
    for b in range(NB):
        j = (ng - 1) * NB + b
        pltpu.make_async_copy(rows.at[b], acc.at[sidx.at[j]], ssem[b]).wait()


def _conv_body(xs_hbm, gth_hbm, sct_hbm, zeros_hbm, out_hbm,
               acc, gidx, sidx, rows, *sems):
    gsem = sems[:NB]
    ssem = sems[NB:]
    c = lax.axis_index("c")
    s = lax.axis_index("s")

    @pl.when(c == 0)
    def _():
        pltpu.sync_copy(zeros_hbm.at[pl.ds(0, RPTA)],
                        acc.at[pl.ds(s * RPTA, RPTA)])
        for half in range(K0 // KH):
            start = s * K0 + half * KH
            pltpu.sync_copy(gth_hbm.at[pl.ds(start, KH)], gidx)
            pltpu.sync_copy(sct_hbm.at[pl.ds(start, KH)], sidx)
            if half == 0:
                plsc.subcore_barrier()
            _conv_pipeline(xs_hbm, acc, gidx, sidx, rows, gsem, ssem, KH)
        plsc.subcore_barrier()
        pltpu.sync_copy(acc.at[pl.ds(s * RPTA, RPTA)],
                        out_hbm.at[pl.ds(s * RPTA, RPTA)])


def _conv_call(xs, gth_idx, sct_idx, zeros128):
    return pl.kernel(
        _conv_body,
        out_type=jax.ShapeDtypeStruct((NP, D), jnp.float32),
        mesh=_MESH,
        scratch_types=[
            pltpu.VMEM_SHARED((NPA, D), jnp.float32),
            pltpu.VMEM((KH, CH), jnp.int32),
            pltpu.VMEM((KH, CH), jnp.int32),
            pltpu.VMEM((NB, CH, D), jnp.float32),
        ] + [pltpu.SemaphoreType.DMA] * (2 * NB),
    )(xs, gth_idx, sct_idx, zeros128)


def _enc_body(x_ref, w_ref, b_ref, deg_ref, y0_ref, xs_ref, dinv_ref):
    h = lax.dot_general(x_ref[...], w_ref[...], (((1,), (1,)), ((), ())),
                        preferred_element_type=jnp.float32)
    y0 = jnp.maximum(h + b_ref[...], 0.0)
    deg = 1.0 + deg_ref[0, :, 0:1] + deg_ref[1, :, 0:1]
    dinv = lax.rsqrt(deg)
    y0_ref[...] = y0
    xs_ref[...] = dinv * y0
    dinv_ref[...] = dinv


def _enc_call(xpad, enc_W, enc_b, degp):
    return pl.pallas_call(
        _enc_body,
        grid=(NP // BN,),
        in_specs=[
            pl.BlockSpec((BN, D), lambda i: (i, 0)),
            pl.BlockSpec((D, D), lambda i: (0, 0)),
            pl.BlockSpec((1, D), lambda i: (0, 0)),
            pl.BlockSpec((NC, BN, D), lambda i: (0, i, 0)),
        ],
        out_specs=[
            pl.BlockSpec((BN, D), lambda i: (i, 0)),
            pl.BlockSpec((BN, D), lambda i: (i, 0)),
            pl.BlockSpec((BN, 1), lambda i: (i, 0)),
        ],
        out_shape=[
            jax.ShapeDtypeStruct((NP, D), jnp.float32),
            jax.ShapeDtypeStruct((NP, D), jnp.float32),
            jax.ShapeDtypeStruct((NP, 1), jnp.float32),
        ],
    )(xpad, enc_W, enc_b.reshape(1, D), degp)


def _layer_body(x_ref, y_ref, g_ref, dinv_ref, w_ref,
                xn_ref, yn_ref, xsn_ref):
    x = x_ref[...]
    y = y_ref[...]
    dinv = dinv_ref[...]
    w = w_ref[...]
    s = dinv * x + g_ref[...]
    yv = dinv * s
    m1 = lax.dot_general(yv, w, (((1,), (0,)), ((), ())),
                         preferred_element_type=jnp.float32)
    m2 = lax.dot_general(yv, w, (((1,), (1,)), ((), ())),
                         preferred_element_type=jnp.float32)
    inter = (0.5 * STEP) * (m1 + m2)
    c = jnp.maximum(x + inter + x, 0.0)
    yn = y + DT * (c - ALPHA * y - GAMMA * x)
    xn = x + DT * yn
    yn_ref[...] = yn
    xn_ref[...] = xn
    xsn_ref[...] = dinv * xn


def _layer_call(X, Y, G, dinv, conv_W):
    return pl.pallas_call(
        _layer_body,
        grid=(NP // BN,),
        in_specs=[
            pl.BlockSpec((BN, D), lambda i: (i, 0)),
            pl.BlockSpec((BN, D), lambda i: (i, 0)),
            pl.BlockSpec((BN, D), lambda i: (i, 0)),
            pl.BlockSpec((BN, 1), lambda i: (i, 0)),
            pl.BlockSpec((D, D), lambda i: (0, 0)),
        ],
        out_specs=[
            pl.BlockSpec((BN, D), lambda i: (i, 0)),
            pl.BlockSpec((BN, D), lambda i: (i, 0)),
            pl.BlockSpec((BN, D), lambda i: (i, 0)),
        ],
        out_shape=[
            jax.ShapeDtypeStruct((NP, D), jnp.float32),
            jax.ShapeDtypeStruct((NP, D), jnp.float32),
            jax.ShapeDtypeStruct((NP, D), jnp.float32),
        ],
    )(X, Y, G, dinv, conv_W)


def _dec_body(x_ref, w_ref, b_ref, out_ref):
    out_ref[...] = lax.dot_general(
        x_ref[...], w_ref[...], (((1,), (1,)), ((), ())),
        preferred_element_type=jnp.float32) + b_ref[...]


def _dec_call(X, dec_W, dec_b):
    return pl.pallas_call(
        _dec_body,
        grid=(NP // BN,),
        in_specs=[
            pl.BlockSpec((BN, D), lambda i: (i, 0)),
            pl.BlockSpec((NCLASS, D), lambda i: (0, 0)),
            pl.BlockSpec((1, NCLASS), lambda i: (0, 0)),
        ],
        out_specs=pl.BlockSpec((BN, NCLASS), lambda i: (i, 0)),
        out_shape=jax.ShapeDtypeStruct((NP, NCLASS), jnp.float32),
    )(X, dec_W, dec_b.reshape(1, NCLASS))


def kernel(x, edge_index, enc_W, enc_b, conv_W, dec_W, dec_b):
    src = edge_index[0]
    dst = edge_index[1]
    pad = jnp.full((EP - E,), DUMMY, dtype=jnp.int32)
    gth_idx = jnp.concatenate([dst, pad]).reshape(EPCH, CH)
    sct_idx = jnp.concatenate([src, pad]).reshape(EPCH, CH)
    xpad = jnp.pad(x, ((0, NP - N), (0, 0)))
    ones128 = jnp.ones((CH, D), jnp.float32)
    zeros128 = jnp.zeros((RPT, D), jnp.float32)

    degp = _deg_call(gth_idx, zeros128, ones128)
    Y, xs, dinv = _enc_call(xpad, enc_W, enc_b, degp)
    X = Y
    for _ in range(NLAYERS):
        G = _conv_call(xs, gth_idx, sct_idx, zeros128)
        X, Y, xs = _layer_call(X, Y, G, dinv, conv_W)
    out = _dec_call(X, dec_W, dec_b)
    return out[:N]


# speedup vs baseline: 2.7619x; 2.7619x over previous
import functools

import jax
import jax.numpy as jnp
from jax import lax
from jax.experimental import pallas as pl
from jax.experimental.pallas import tpu as pltpu
from jax.experimental.pallas import tpu_sc as plsc

N = 10000
E = 160000
D = 128
NCLASS = 16
NLAYERS = 4
STEP = 1.0
DT = 1.0
ALPHA = 1.0
GAMMA = 1.0

NC = 2
NS = 16
NW = NC * NS
CH = 128
NCH = 40
EP = NW * NCH * CH
EPCH = EP // CH
NPA = 10112
RPTA = NPA // NS
NDUM = NPA - N
DUMMY = 10000
NP = 10240
BN = 1024
RPT = NP // NS

_MESH = plsc.VectorSubcoreMesh(
    core_axis_name="c", subcore_axis_name="s", num_cores=NC, num_subcores=NS)


def _deg_body(dst_hbm, zeros_hbm, ones_hbm, out_hbm, acc, idx_v, ones_v, sem):
    c = lax.axis_index("c")
    s = lax.axis_index("s")
    wid = s * NC + c
    pltpu.sync_copy(zeros_hbm, acc.at[pl.ds(s * RPT, RPT)])
    pltpu.sync_copy(ones_hbm, ones_v)
    pltpu.sync_copy(dst_hbm.at[pl.ds(wid * NCH, NCH)], idx_v)
    plsc.subcore_barrier()

    def fire(j, carry):
        pltpu.async_copy(ones_v, acc.at[idx_v.at[j]], sem, add=True)
        return carry

    lax.fori_loop(0, NCH, fire, 0)

    def drain(j, carry):
        pltpu.make_async_copy(ones_v, acc.at[idx_v.at[j]], sem).wait()
        return carry

    lax.fori_loop(0, NCH, drain, 0)
    plsc.subcore_barrier()
    pltpu.sync_copy(acc.at[pl.ds(s * RPT, RPT)],
                    out_hbm.at[c].at[pl.ds(s * RPT, RPT)])


def _deg_call(dst_idx, zeros128, ones128):
    return pl.kernel(
        _deg_body,
        out_type=jax.ShapeDtypeStruct((NC, NP, D), jnp.float32),
        mesh=_MESH,
        scratch_types=[
            pltpu.VMEM_SHARED((NP, D), jnp.float32),
            pltpu.VMEM((NCH, CH), jnp.int32),
            pltpu.VMEM((CH, D), jnp.float32),
            pltpu.SemaphoreType.DMA,
        ],
    )(dst_idx, zeros128, ones128)


NB = 2


def _conv_pipeline(xs_hbm, acc, gidx, sidx, rows, gsem, ssem, K):
    ng = K // NB
    for b in range(NB):
        pltpu.async_copy(xs_hbm.at[gidx.at[b]], rows.at[b], gsem[b])

    def group(g, carry):
        for b in range(NB):
            j = g * NB + b
            pltpu.make_async_copy(
                xs_hbm.at[gidx.at[j]], rows.at[b], gsem[b]).wait()
            pltpu.async_copy(rows.at[b], acc.at[sidx.at[j]], ssem[b], add=True)
        jn = (g + 1) * NB
        for b in range(NB):
            j = g * NB + b
            pltpu.make_async_copy(
                rows.at[b], acc.at[sidx.at[j]], ssem[b]).wait()
            pltpu.async_copy(xs_hbm.at[gidx.at[jn + b]], rows.at[b], gsem[b])
        return carry

    lax.fori_loop(0, ng - 1, group, 0)

    for b in range(NB):
        j = (ng - 1) * NB + b
        pltpu.make_async_copy(xs_hbm.at[gidx.at[j]], rows.at[b], gsem[b]).wait()
        pltpu.async_copy(rows.at[b], acc.at[sidx.at[j]], ssem[b], add=True)
    for b in range(NB):
        j = (ng - 1) * NB + b
        pltpu.make_async_copy(rows.at[b], acc.at[sidx.at[j]], ssem[b]).wait()


def _conv_body(xs_hbm, gth_hbm, sct_hbm, zeros_hbm, out_hbm,
               acc, gidx, sidx, rows, *sems):
    gsem = sems[:NB]
    ssem = sems[NB:]
    c = lax.axis_index("c")
    s = lax.axis_index("s")
    wid = c * NS + s
    pltpu.sync_copy(zeros_hbm.at[pl.ds(0, RPTA)],
                    acc.at[pl.ds(s * RPTA, RPTA)])
    pltpu.sync_copy(gth_hbm.at[pl.ds(wid * NCH, NCH)], gidx)
    pltpu.sync_copy(sct_hbm.at[pl.ds(wid * NCH, NCH)], sidx)
    plsc.subcore_barrier()
    _conv_pipeline(xs_hbm, acc, gidx, sidx, rows, gsem, ssem, NCH)
    plsc.subcore_barrier()
    pltpu.sync_copy(acc.at[pl.ds(s * RPTA, RPTA)],
                    out_hbm.at[c].at[pl.ds(s * RPTA, RPTA)])


def _conv_call(xs, gth_idx, sct_idx, zeros128):
    return pl.kernel(
        _conv_body,
        out_type=jax.ShapeDtypeStruct((NC, NP, D), jnp.float32),
        mesh=_MESH,
        scratch_types=[
            pltpu.VMEM_SHARED((NPA, D), jnp.float32),
            pltpu.VMEM((NCH, CH), jnp.int32),
            pltpu.VMEM((NCH, CH), jnp.int32),
            pltpu.VMEM((NB, CH, D), jnp.float32),
        ] + [pltpu.SemaphoreType.DMA] * (2 * NB),
    )(xs, gth_idx, sct_idx, zeros128)


def _enc_body(x_ref, w_ref, b_ref, deg_ref, y0_ref, xs_ref, dinv_ref):
    h = lax.dot_general(x_ref[...], w_ref[...], (((1,), (1,)), ((), ())),
                        preferred_element_type=jnp.float32)
    y0 = jnp.maximum(h + b_ref[...], 0.0)
    deg = 1.0 + deg_ref[0, :, 0:1] + deg_ref[1, :, 0:1]
    dinv = lax.rsqrt(deg)
    y0_ref[...] = y0
    xs_ref[...] = dinv * y0
    dinv_ref[...] = dinv


def _enc_call(xpad, enc_W, enc_b, degp):
    return pl.pallas_call(
        _enc_body,
        grid=(NP // BN,),
        in_specs=[
            pl.BlockSpec((BN, D), lambda i: (i, 0)),
            pl.BlockSpec((D, D), lambda i: (0, 0)),
            pl.BlockSpec((1, D), lambda i: (0, 0)),
            pl.BlockSpec((NC, BN, D), lambda i: (0, i, 0)),
        ],
        out_specs=[
            pl.BlockSpec((BN, D), lambda i: (i, 0)),
            pl.BlockSpec((BN, D), lambda i: (i, 0)),
            pl.BlockSpec((BN, 1), lambda i: (i, 0)),
        ],
        out_shape=[
            jax.ShapeDtypeStruct((NP, D), jnp.float32),
            jax.ShapeDtypeStruct((NP, D), jnp.float32),
            jax.ShapeDtypeStruct((NP, 1), jnp.float32),
        ],
    )(xpad, enc_W, enc_b.reshape(1, D), degp)


def _layer_body(x_ref, y_ref, g_ref, dinv_ref, w_ref,
                xn_ref, yn_ref, xsn_ref):
    x = x_ref[...]
    y = y_ref[...]
    dinv = dinv_ref[...]
    w = w_ref[...]
    s = dinv * x + g_ref[0] + g_ref[1]
    yv = dinv * s
    m1 = lax.dot_general(yv, w, (((1,), (0,)), ((), ())),
                         preferred_element_type=jnp.float32)
    m2 = lax.dot_general(yv, w, (((1,), (1,)), ((), ())),
                         preferred_element_type=jnp.float32)
    inter = (0.5 * STEP) * (m1 + m2)
    c = jnp.maximum(x + inter + x, 0.0)
    yn = y + DT * (c - ALPHA * y - GAMMA * x)
    xn = x + DT * yn
    yn_ref[...] = yn
    xn_ref[...] = xn
    xsn_ref[...] = dinv * xn


def _layer_call(X, Y, G, dinv, conv_W):
    return pl.pallas_call(
        _layer_body,
        grid=(NP // BN,),
        in_specs=[
            pl.BlockSpec((BN, D), lambda i: (i, 0)),
            pl.BlockSpec((BN, D), lambda i: (i, 0)),
            pl.BlockSpec((NC, BN, D), lambda i: (0, i, 0)),
            pl.BlockSpec((BN, 1), lambda i: (i, 0)),
            pl.BlockSpec((D, D), lambda i: (0, 0)),
        ],
        out_specs=[
            pl.BlockSpec((BN, D), lambda i: (i, 0)),
            pl.BlockSpec((BN, D), lambda i: (i, 0)),
            pl.BlockSpec((BN, D), lambda i: (i, 0)),
        ],
        out_shape=[
            jax.ShapeDtypeStruct((NP, D), jnp.float32),
            jax.ShapeDtypeStruct((NP, D), jnp.float32),
            jax.ShapeDtypeStruct((NP, D), jnp.float32),
        ],
    )(X, Y, G, dinv, conv_W)


def _dec_body(x_ref, w_ref, b_ref, out_ref):
    out_ref[...] = lax.dot_general(
        x_ref[...], w_ref[...], (((1,), (1,)), ((), ())),
        preferred_element_type=jnp.float32) + b_ref[...]


def _dec_call(X, dec_W, dec_b):
    return pl.pallas_call(
        _dec_body,
        grid=(NP // BN,),
        in_specs=[
            pl.BlockSpec((BN, D), lambda i: (i, 0)),
            pl.BlockSpec((NCLASS, D), lambda i: (0, 0)),
            pl.BlockSpec((1, NCLASS), lambda i: (0, 0)),
        ],
        out_specs=pl.BlockSpec((BN, NCLASS), lambda i: (i, 0)),
        out_shape=jax.ShapeDtypeStruct((NP, NCLASS), jnp.float32),
    )(X, dec_W, dec_b.reshape(1, NCLASS))


def kernel(x, edge_index, enc_W, enc_b, conv_W, dec_W, dec_b):
    src = edge_index[0]
    dst = edge_index[1]
    pad = N + jnp.arange(EP - E, dtype=jnp.int32) % NDUM
    gth_idx = jnp.concatenate([dst, pad]).reshape(EPCH, CH)
    sct_idx = jnp.concatenate([src, pad]).reshape(EPCH, CH)
    xpad = jnp.pad(x, ((0, NP - N), (0, 0)))
    ones128 = jnp.ones((CH, D), jnp.float32)
    zeros128 = jnp.zeros((RPT, D), jnp.float32)

    degp = _deg_call(gth_idx, zeros128, ones128)
    Y, xs, dinv = _enc_call(xpad, enc_W, enc_b, degp)
    X = Y
    for _ in range(NLAYERS):
        G = _conv_call(xs, gth_idx, sct_idx, zeros128)
        X, Y, xs = _layer_call(X, Y, G, dinv, conv_W)
    out = _dec_call(X, dec_W, dec_b)
    return out[:N]


# speedup vs baseline: 2.8375x; 1.0274x over previous
import functools

import jax
import jax.numpy as jnp
from jax import lax
from jax.experimental import pallas as pl
from jax.experimental.pallas import tpu as pltpu
from jax.experimental.pallas import tpu_sc as plsc

N = 10000
E = 160000
D = 128
NCLASS = 16
NLAYERS = 4
STEP = 1.0
DT = 1.0
ALPHA = 1.0
GAMMA = 1.0

NC = 2
NS = 16
NW = NC * NS
CH = 128
NCH = 40
EP = NW * NCH * CH
EPCH = EP // CH
NPA = 10112
RPTA = NPA // NS
NDUM = NPA - N
DUMMY = 10000
NP = 10240
BN = 1024
RPT = NP // NS

_MESH = plsc.VectorSubcoreMesh(
    core_axis_name="c", subcore_axis_name="s", num_cores=NC, num_subcores=NS)


def _deg_body(dst_hbm, zeros_hbm, ones_hbm, out_hbm, acc, idx_v, ones_v, sem):
    c = lax.axis_index("c")
    s = lax.axis_index("s")
    wid = s * NC + c
    pltpu.sync_copy(zeros_hbm, acc.at[pl.ds(s * RPT, RPT)])
    pltpu.sync_copy(ones_hbm, ones_v)
    pltpu.sync_copy(dst_hbm.at[pl.ds(wid * NCH, NCH)], idx_v)
    plsc.subcore_barrier()

    def fire(j, carry):
        pltpu.async_copy(ones_v, acc.at[idx_v.at[j]], sem, add=True)
        return carry

    lax.fori_loop(0, NCH, fire, 0)

    def drain(j, carry):
        pltpu.make_async_copy(ones_v, acc.at[idx_v.at[j]], sem).wait()
        return carry

    lax.fori_loop(0, NCH, drain, 0)
    plsc.subcore_barrier()
    pltpu.sync_copy(acc.at[pl.ds(s * RPT, RPT)],
                    out_hbm.at[c].at[pl.ds(s * RPT, RPT)])


def _deg_call(dst_idx, zeros128, ones128):
    return pl.kernel(
        _deg_body,
        out_type=jax.ShapeDtypeStruct((NC, NP, D), jnp.float32),
        mesh=_MESH,
        scratch_types=[
            pltpu.VMEM_SHARED((NP, D), jnp.float32),
            pltpu.VMEM((NCH, CH), jnp.int32),
            pltpu.VMEM((CH, D), jnp.float32),
            pltpu.SemaphoreType.DMA,
        ],
    )(dst_idx, zeros128, ones128)


NB = 2


def _conv_pipeline(xs_hbm, acc, gidx, sidx, rows, gsem, ssem, K):
    ng = K // NB
    for b in range(NB):
        pltpu.async_copy(xs_hbm.at[gidx.at[b]], rows.at[b], gsem[b])

    def group(g, carry):
        for b in range(NB):
            j = g * NB + b
            pltpu.make_async_copy(
                xs_hbm.at[gidx.at[j]], rows.at[b], gsem[b]).wait()
            pltpu.async_copy(rows.at[b], acc.at[sidx.at[j]], ssem[b], add=True)
        jn = (g + 1) * NB
        for b in range(NB):
            j = g * NB + b
            pltpu.make_async_copy(
                rows.at[b], acc.at[sidx.at[j]], ssem[b]).wait()
            pltpu.async_copy(xs_hbm.at[gidx.at[jn + b]], rows.at[b], gsem[b])
        return carry

    lax.fori_loop(0, ng - 1, group, 0)

    for b in range(NB):
        j = (ng - 1) * NB + b
        pltpu.make_async_copy(xs_hbm.at[gidx.at[j]], rows.at[b], gsem[b]).wait()
        pltpu.async_copy(rows.at[b], acc.at[sidx.at[j]], ssem[b], add=True)
    for b in range(NB):
        j = (ng - 1) * NB + b
        pltpu.make_async_copy(rows.at[b], acc.at[sidx.at[j]], ssem[b]).wait()


def _conv_body(xs_hbm, gth_hbm, sct_hbm, zeros_hbm, out_hbm,
               acc, gidx, sidx, rows, *sems):
    gsem = sems[:NB]
    ssem = sems[NB:]
    c = lax.axis_index("c")
    s = lax.axis_index("s")
    wid = c * NS + s
    pltpu.sync_copy(zeros_hbm.at[pl.ds(0, RPTA)],
                    acc.at[pl.ds(s * RPTA, RPTA)])
    pltpu.sync_copy(gth_hbm.at[pl.ds(wid * NCH, NCH)], gidx)
    pltpu.sync_copy(sct_hbm.at[pl.ds(wid * NCH, NCH)], sidx)
    plsc.subcore_barrier()
    _conv_pipeline(xs_hbm, acc, gidx, sidx, rows, gsem, ssem, NCH)
    plsc.subcore_barrier()
    pltpu.sync_copy(acc.at[pl.ds(s * RPTA, RPTA)],
                    out_hbm.at[c].at[pl.ds(s * RPTA, RPTA)])


def _conv_call(xs, gth_idx, sct_idx, zeros128):
    return pl.kernel(
        _conv_body,
        out_type=jax.ShapeDtypeStruct((NC, NP, D), jnp.float32),
        mesh=_MESH,
        scratch_types=[
            pltpu.VMEM_SHARED((NPA, D), jnp.float32),
            pltpu.VMEM((NCH, CH), jnp.int32),
            pltpu.VMEM((NCH, CH), jnp.int32),
            pltpu.VMEM((NB, CH, D), jnp.float32),
        ] + [pltpu.SemaphoreType.DMA] * (2 * NB),
    )(xs, gth_idx, sct_idx, zeros128)


def _enc_body(x_ref, w_ref, b_ref, deg_ref, y0_ref, xs_ref, dinv_ref):
    h = lax.dot_general(x_ref[...], w_ref[...], (((1,), (1,)), ((), ())),
                        preferred_element_type=jnp.float32)
    y0 = jnp.maximum(h + b_ref[...], 0.0)
    deg = 1.0 + deg_ref[0, :, 0:1] + deg_ref[1, :, 0:1]
    dinv = lax.rsqrt(deg)
    y0_ref[...] = y0
    xs_ref[...] = dinv * y0
    dinv_ref[...] = dinv


def _enc_call(xpad, enc_W, enc_b, degp):
    return pl.pallas_call(
        _enc_body,
        grid=(NP // BN,),
        in_specs=[
            pl.BlockSpec((BN, D), lambda i: (i, 0)),
            pl.BlockSpec((D, D), lambda i: (0, 0)),
            pl.BlockSpec((1, D), lambda i: (0, 0)),
            pl.BlockSpec((NC, BN, D), lambda i: (0, i, 0)),
        ],
        out_specs=[
            pl.BlockSpec((BN, D), lambda i: (i, 0)),
            pl.BlockSpec((BN, D), lambda i: (i, 0)),
            pl.BlockSpec((BN, 1), lambda i: (i, 0)),
        ],
        out_shape=[
            jax.ShapeDtypeStruct((NP, D), jnp.float32),
            jax.ShapeDtypeStruct((NP, D), jnp.float32),
            jax.ShapeDtypeStruct((NP, 1), jnp.float32),
        ],
    )(xpad, enc_W, enc_b.reshape(1, D), degp)


def _layer_body(x_ref, y_ref, g_ref, dinv_ref, w_ref,
                xn_ref, yn_ref, xsn_ref):
    x = x_ref[...]
    y = y_ref[...]
    dinv = dinv_ref[...]
    w = w_ref[...]
    s = dinv * x + g_ref[0] + g_ref[1]
    yv = dinv * s
    m1 = lax.dot_general(yv, w, (((1,), (0,)), ((), ())),
                         preferred_element_type=jnp.float32)
    m2 = lax.dot_general(yv, w, (((1,), (1,)), ((), ())),
                         preferred_element_type=jnp.float32)
    inter = (0.5 * STEP) * (m1 + m2)
    c = jnp.maximum(x + inter + x, 0.0)
    yn = y + DT * (c - ALPHA * y - GAMMA * x)
    xn = x + DT * yn
    yn_ref[...] = yn
    xn_ref[...] = xn
    xsn_ref[...] = dinv * xn


def _layer_call(X, Y, G, dinv, conv_W):
    return pl.pallas_call(
        _layer_body,
        grid=(NP // BN,),
        in_specs=[
            pl.BlockSpec((BN, D), lambda i: (i, 0)),
            pl.BlockSpec((BN, D), lambda i: (i, 0)),
            pl.BlockSpec((NC, BN, D), lambda i: (0, i, 0)),
            pl.BlockSpec((BN, 1), lambda i: (i, 0)),
            pl.BlockSpec((D, D), lambda i: (0, 0)),
        ],
        out_specs=[
            pl.BlockSpec((BN, D), lambda i: (i, 0)),
            pl.BlockSpec((BN, D), lambda i: (i, 0)),
            pl.BlockSpec((BN, D), lambda i: (i, 0)),
        ],
        out_shape=[
            jax.ShapeDtypeStruct((NP, D), jnp.float32),
            jax.ShapeDtypeStruct((NP, D), jnp.float32),
            jax.ShapeDtypeStruct((NP, D), jnp.float32),
        ],
    )(X, Y, G, dinv, conv_W)


def _last_body(x_ref, y_ref, g_ref, dinv_ref, w_ref, dw_ref, db_ref, out_ref):
    x = x_ref[...]
    y = y_ref[...]
    dinv = dinv_ref[...]
    w = w_ref[...]
    s = dinv * x + g_ref[0] + g_ref[1]
    yv = dinv * s
    m1 = lax.dot_general(yv, w, (((1,), (0,)), ((), ())),
                         preferred_element_type=jnp.float32)
    m2 = lax.dot_general(yv, w, (((1,), (1,)), ((), ())),
                         preferred_element_type=jnp.float32)
    inter = (0.5 * STEP) * (m1 + m2)
    c = jnp.maximum(x + inter + x, 0.0)
    yn = y + DT * (c - ALPHA * y - GAMMA * x)
    xn = x + DT * yn
    out_ref[...] = lax.dot_general(
        xn, dw_ref[...], (((1,), (1,)), ((), ())),
        preferred_element_type=jnp.float32) + db_ref[...]


def _last_call(X, Y, G, dinv, conv_W, dec_W, dec_b):
    return pl.pallas_call(
        _last_body,
        grid=(NP // BN,),
        in_specs=[
            pl.BlockSpec((BN, D), lambda i: (i, 0)),
            pl.BlockSpec((BN, D), lambda i: (i, 0)),
            pl.BlockSpec((NC, BN, D), lambda i: (0, i, 0)),
            pl.BlockSpec((BN, 1), lambda i: (i, 0)),
            pl.BlockSpec((D, D), lambda i: (0, 0)),
            pl.BlockSpec((NCLASS, D), lambda i: (0, 0)),
            pl.BlockSpec((1, NCLASS), lambda i: (0, 0)),
        ],
        out_specs=pl.BlockSpec((BN, NCLASS), lambda i: (i, 0)),
        out_shape=jax.ShapeDtypeStruct((NP, NCLASS), jnp.float32),
    )(X, Y, G, dinv, conv_W, dec_W, dec_b.reshape(1, NCLASS))


def kernel(x, edge_index, enc_W, enc_b, conv_W, dec_W, dec_b):
    src = edge_index[0]
    dst = edge_index[1]
    pad = N + jnp.arange(EP - E, dtype=jnp.int32) % NDUM
    gth_idx = jnp.concatenate([dst, pad]).reshape(EPCH, CH)
    sct_idx = jnp.concatenate([src, pad]).reshape(EPCH, CH)
    xpad = jnp.pad(x, ((0, NP - N), (0, 0)))
    ones128 = jnp.ones((CH, D), jnp.float32)
    zeros128 = jnp.zeros((RPT, D), jnp.float32)

    degp = _deg_call(gth_idx, zeros128, ones128)
    Y, xs, dinv = _enc_call(xpad, enc_W, enc_b, degp)
    X = Y
    for _ in range(NLAYERS - 1):
        G = _conv_call(xs, gth_idx, sct_idx, zeros128)
        X, Y, xs = _layer_call(X, Y, G, dinv, conv_W)
    G = _conv_call(xs, gth_idx, sct_idx, zeros128)
    out = _last_call(X, Y, G, dinv, conv_W, dec_W, dec_b)
    return out[:N]


# speedup vs baseline: 3.1396x; 1.1065x over previous
import functools

import jax
import jax.numpy as jnp
from jax import lax
from jax.experimental import pallas as pl
from jax.experimental.pallas import tpu as pltpu
from jax.experimental.pallas import tpu_sc as plsc

N = 10000
E = 160000
D = 128
NCLASS = 16
NLAYERS = 4
STEP = 1.0
DT = 1.0
ALPHA = 1.0
GAMMA = 1.0

NC = 2
NS = 16
NW = NC * NS
CH = 128
NCH = 40
EP = NW * NCH * CH
EPCH = EP // CH
NPA = 10112
RPTA = NPA // NS
NDUM = NPA - N
DUMMY = 10000
NP = 10240
BN = 1024
RPT = NP // NS

_MESH = plsc.VectorSubcoreMesh(
    core_axis_name="c", subcore_axis_name="s", num_cores=NC, num_subcores=NS)


def _deg_body(dst_hbm, zeros_hbm, ones_hbm, out_hbm, acc, idx_v, ones_v, sem):
    c = lax.axis_index("c")
    s = lax.axis_index("s")
    wid = s * NC + c
    pltpu.sync_copy(zeros_hbm, acc.at[pl.ds(s * RPT, RPT)])
    pltpu.sync_copy(ones_hbm, ones_v)
    pltpu.sync_copy(dst_hbm.at[pl.ds(wid * NCH, NCH)], idx_v)
    plsc.subcore_barrier()

    def fire(j, carry):
        pltpu.async_copy(ones_v, acc.at[idx_v.at[j]], sem, add=True)
        return carry

    lax.fori_loop(0, NCH, fire, 0)

    def drain(j, carry):
        pltpu.make_async_copy(ones_v, acc.at[idx_v.at[j]], sem).wait()
        return carry

    lax.fori_loop(0, NCH, drain, 0)
    plsc.subcore_barrier()
    pltpu.sync_copy(acc.at[pl.ds(s * RPT, RPT)],
                    out_hbm.at[c].at[pl.ds(s * RPT, RPT)])


def _deg_call(dst_idx, zeros128, ones128):
    return pl.kernel(
        _deg_body,
        out_type=jax.ShapeDtypeStruct((NC, NP, D), jnp.float32),
        mesh=_MESH,
        scratch_types=[
            pltpu.VMEM_SHARED((NP, D), jnp.float32),
            pltpu.VMEM((NCH, CH), jnp.int32),
            pltpu.VMEM((CH, D), jnp.float32),
            pltpu.SemaphoreType.DMA,
        ],
    )(dst_idx, zeros128, ones128)


NB = 4
CCH = 64
KC = EP // (NW * CCH)
KCH = KC // 2


def _conv_pipeline(xs_hbm, acc, gidx, sidx, rows, gsem, ssem, K):
    ng = K // NB
    for b in range(NB):
        pltpu.async_copy(xs_hbm.at[gidx.at[b]], rows.at[b], gsem[b])

    def group(g, carry):
        for b in range(NB):
            j = g * NB + b
            pltpu.make_async_copy(
                xs_hbm.at[gidx.at[j]], rows.at[b], gsem[b]).wait()
            pltpu.async_copy(rows.at[b], acc.at[sidx.at[j]], ssem[b], add=True)
        jn = (g + 1) * NB
        for b in range(NB):
            j = g * NB + b
            pltpu.make_async_copy(
                rows.at[b], acc.at[sidx.at[j]], ssem[b]).wait()
            pltpu.async_copy(xs_hbm.at[gidx.at[jn + b]], rows.at[b], gsem[b])
        return carry

    lax.fori_loop(0, ng - 1, group, 0)

    for b in range(NB):
        j = (ng - 1) * NB + b
        pltpu.make_async_copy(xs_hbm.at[gidx.at[j]], rows.at[b], gsem[b]).wait()
        pltpu.async_copy(rows.at[b], acc.at[sidx.at[j]], ssem[b], add=True)
    for b in range(NB):
        j = (ng - 1) * NB + b
        pltpu.make_async_copy(rows.at[b], acc.at[sidx.at[j]], ssem[b]).wait()


def _conv_body(xs_hbm, gth_hbm, sct_hbm, zeros_hbm, out_hbm,
               acc, gidx, sidx, rows, *sems):
    gsem = sems[:NB]
    ssem = sems[NB:]
    c = lax.axis_index("c")
    s = lax.axis_index("s")
    wid = c * NS + s
    pltpu.sync_copy(zeros_hbm.at[pl.ds(0, RPTA)],
                    acc.at[pl.ds(s * RPTA, RPTA)])
    for half in range(KC // KCH):
        start = wid * KC + half * KCH
        pltpu.sync_copy(gth_hbm.at[pl.ds(start, KCH)], gidx)
        pltpu.sync_copy(sct_hbm.at[pl.ds(start, KCH)], sidx)
        if half == 0:
            plsc.subcore_barrier()
        _conv_pipeline(xs_hbm, acc, gidx, sidx, rows, gsem, ssem, KCH)
    plsc.subcore_barrier()
    pltpu.sync_copy(acc.at[pl.ds(s * RPTA, RPTA)],
                    out_hbm.at[c].at[pl.ds(s * RPTA, RPTA)])


def _conv_call(xs, gth_idx, sct_idx, zeros128):
    return pl.kernel(
        _conv_body,
        out_type=jax.ShapeDtypeStruct((NC, NP, D), jnp.float32),
        mesh=_MESH,
        scratch_types=[
            pltpu.VMEM_SHARED((NPA, D), jnp.float32),
            pltpu.VMEM((KCH, CCH), jnp.int32),
            pltpu.VMEM((KCH, CCH), jnp.int32),
            pltpu.VMEM((NB, CCH, D), jnp.float32),
        ] + [pltpu.SemaphoreType.DMA] * (2 * NB),
    )(xs, gth_idx, sct_idx, zeros128)


def _enc_body(x_ref, w_ref, b_ref, deg_ref, y0_ref, xs_ref, dinv_ref):
    h = lax.dot_general(x_ref[...], w_ref[...], (((1,), (1,)), ((), ())),
                        preferred_element_type=jnp.float32)
    y0 = jnp.maximum(h + b_ref[...], 0.0)
    deg = 1.0 + deg_ref[0, :, 0:1] + deg_ref[1, :, 0:1]
    dinv = lax.rsqrt(deg)
    y0_ref[...] = y0
    xs_ref[...] = dinv * y0
    dinv_ref[...] = dinv


def _enc_call(xpad, enc_W, enc_b, degp):
    return pl.pallas_call(
        _enc_body,
        grid=(NP // BN,),
        in_specs=[
            pl.BlockSpec((BN, D), lambda i: (i, 0)),
            pl.BlockSpec((D, D), lambda i: (0, 0)),
            pl.BlockSpec((1, D), lambda i: (0, 0)),
            pl.BlockSpec((NC, BN, D), lambda i: (0, i, 0)),
        ],
        out_specs=[
            pl.BlockSpec((BN, D), lambda i: (i, 0)),
            pl.BlockSpec((BN, D), lambda i: (i, 0)),
            pl.BlockSpec((BN, 1), lambda i: (i, 0)),
        ],
        out_shape=[
            jax.ShapeDtypeStruct((NP, D), jnp.float32),
            jax.ShapeDtypeStruct((NP, D), jnp.float32),
            jax.ShapeDtypeStruct((NP, 1), jnp.float32),
        ],
    )(xpad, enc_W, enc_b.reshape(1, D), degp)


def _layer_body(x_ref, y_ref, g_ref, dinv_ref, w_ref,
                xn_ref, yn_ref, xsn_ref):
    x = x_ref[...]
    y = y_ref[...]
    dinv = dinv_ref[...]
    w = w_ref[...]
    s = dinv * x + g_ref[0] + g_ref[1]
    yv = dinv * s
    m1 = lax.dot_general(yv, w, (((1,), (0,)), ((), ())),
                         preferred_element_type=jnp.float32)
    m2 = lax.dot_general(yv, w, (((1,), (1,)), ((), ())),
                         preferred_element_type=jnp.float32)
    inter = (0.5 * STEP) * (m1 + m2)
    c = jnp.maximum(x + inter + x, 0.0)
    yn = y + DT * (c - ALPHA * y - GAMMA * x)
    xn = x + DT * yn
    yn_ref[...] = yn
    xn_ref[...] = xn
    xsn_ref[...] = dinv * xn


def _layer_call(X, Y, G, dinv, conv_W):
    return pl.pallas_call(
        _layer_body,
        grid=(NP // BN,),
        in_specs=[
            pl.BlockSpec((BN, D), lambda i: (i, 0)),
            pl.BlockSpec((BN, D), lambda i: (i, 0)),
            pl.BlockSpec((NC, BN, D), lambda i: (0, i, 0)),
            pl.BlockSpec((BN, 1), lambda i: (i, 0)),
            pl.BlockSpec((D, D), lambda i: (0, 0)),
        ],
        out_specs=[
            pl.BlockSpec((BN, D), lambda i: (i, 0)),
            pl.BlockSpec((BN, D), lambda i: (i, 0)),
            pl.BlockSpec((BN, D), lambda i: (i, 0)),
        ],
        out_shape=[
            jax.ShapeDtypeStruct((NP, D), jnp.float32),
            jax.ShapeDtypeStruct((NP, D), jnp.float32),
            jax.ShapeDtypeStruct((NP, D), jnp.float32),
        ],
    )(X, Y, G, dinv, conv_W)


def _last_body(x_ref, y_ref, g_ref, dinv_ref, w_ref, dw_ref, db_ref, out_ref):
    x = x_ref[...]
    y = y_ref[...]
    dinv = dinv_ref[...]
    w = w_ref[...]
    s = dinv * x + g_ref[0] + g_ref[1]
    yv = dinv * s
    m1 = lax.dot_general(yv, w, (((1,), (0,)), ((), ())),
                         preferred_element_type=jnp.float32)
    m2 = lax.dot_general(yv, w, (((1,), (1,)), ((), ())),
                         preferred_element_type=jnp.float32)
    inter = (0.5 * STEP) * (m1 + m2)
    c = jnp.maximum(x + inter + x, 0.0)
    yn = y + DT * (c - ALPHA * y - GAMMA * x)
    xn = x + DT * yn
    out_ref[...] = lax.dot_general(
        xn, dw_ref[...], (((1,), (1,)), ((), ())),
        preferred_element_type=jnp.float32) + db_ref[...]


def _last_call(X, Y, G, dinv, conv_W, dec_W, dec_b):
    return pl.pallas_call(
        _last_body,
        grid=(NP // BN,),
        in_specs=[
            pl.BlockSpec((BN, D), lambda i: (i, 0)),
            pl.BlockSpec((BN, D), lambda i: (i, 0)),
            pl.BlockSpec((NC, BN, D), lambda i: (0, i, 0)),
            pl.BlockSpec((BN, 1), lambda i: (i, 0)),
            pl.BlockSpec((D, D), lambda i: (0, 0)),
            pl.BlockSpec((NCLASS, D), lambda i: (0, 0)),
            pl.BlockSpec((1, NCLASS), lambda i: (0, 0)),
        ],
        out_specs=pl.BlockSpec((BN, NCLASS), lambda i: (i, 0)),
        out_shape=jax.ShapeDtypeStruct((NP, NCLASS), jnp.float32),
    )(X, Y, G, dinv, conv_W, dec_W, dec_b.reshape(1, NCLASS))


def kernel(x, edge_index, enc_W, enc_b, conv_W, dec_W, dec_b):
    src = edge_index[0]
    dst = edge_index[1]
    pad = N + jnp.arange(EP - E, dtype=jnp.int32) % NDUM
    gth_flat = jnp.concatenate([dst, pad])
    sct_flat = jnp.concatenate([src, pad])
    gth_idx = gth_flat.reshape(EPCH, CH)
    sct_idx = sct_flat.reshape(EPCH, CH)
    gth_c = gth_flat.reshape(EP // CCH, CCH)
    sct_c = sct_flat.reshape(EP // CCH, CCH)
    xpad = jnp.pad(x, ((0, NP - N), (0, 0)))
    ones128 = jnp.ones((CH, D), jnp.float32)
    zeros128 = jnp.zeros((RPT, D), jnp.float32)

    degp = _deg_call(gth_idx, zeros128, ones128)
    Y, xs, dinv = _enc_call(xpad, enc_W, enc_b, degp)
    X = Y
    for _ in range(NLAYERS - 1):
        G = _conv_call(xs, gth_c, sct_c, zeros128)
        X, Y, xs = _layer_call(X, Y, G, dinv, conv_W)
    G = _conv_call(xs, gth_c, sct_c, zeros128)
    out = _last_call(X, Y, G, dinv, conv_W, dec_W, dec_b)
    return out[:N]
